# baseline (speedup 1.0000x reference)
import functools

import jax
import jax.numpy as jnp
from jax import lax
from jax.experimental import pallas as pl
from jax.experimental.pallas import tpu as pltpu
from jax.experimental.pallas import tpu_sc as plsc

T = 16
NW = 32
LANES = 16
C = 8


def _ztab_body(x_ref, w_ref, b_ref, o_ref):
    acc = jnp.dot(x_ref[...], w_ref[...], preferred_element_type=jnp.float32)
    o_ref[...] = jnp.maximum(acc + b_ref[...], 0.0)


def _z_table(h, q0wt, q0b):
    n, d = h.shape
    dh = q0wt.shape[1]
    blk = 1000
    return pl.pallas_call(
        _ztab_body,
        grid=(n // blk,),
        in_specs=[
            pl.BlockSpec((blk, d), lambda i: (i, 0)),
            pl.BlockSpec((d, dh), lambda i: (0, 0)),
            pl.BlockSpec((1, dh), lambda i: (0, 0)),
        ],
        out_specs=pl.BlockSpec((blk, dh), lambda i: (i, 0)),
        out_shape=jax.ShapeDtypeStruct((n, dh), jnp.float32),
    )(h, q0wt, q0b)


def _sc_body(ns_ref, nbn_ref, nbw_ref, h_ref, z_ref,
             selfh_s, agg_s, selfh_n, agg_n, nbw_out,
             sid, nbid, swt,
             xnb0, xnb1, xwt0, xwt1, xself0, xself1,
             zidx0, zidx1, zbuf0, zbuf1, aggbuf0, aggbuf1, sidx,
             semnb0, semnb1, semz0, semz1, semo0, semo1):
    nb = ns_ref.shape[0]
    pw = nb // NW
    dh = z_ref.shape[1]
    nd = dh // LANES
    wid = lax.axis_index("s") * 2 + lax.axis_index("c")
    base = wid * pw

    xnb = (xnb0, xnb1)
    xwt = (xwt0, xwt1)
    xself = (xself0, xself1)
    zidx = (zidx0, zidx1)
    zbuf = (zbuf0, zbuf1)
    aggbuf = (aggbuf0, aggbuf1)
    semnb = (semnb0, semnb1)
    semz = (semz0, semz1)
    semo = (semo0, semo1)

    pltpu.sync_copy(ns_ref.at[pl.ds(base, pw)], sid)
    pltpu.async_copy(nbn_ref.at[sid], nbid, semnb0).wait()
    pltpu.async_copy(nbw_ref.at[sid], swt, semnb0).wait()
    pltpu.sync_copy(swt, nbw_out.at[pl.ds(base, pw)])

    def fire_z(b):
        return pltpu.async_copy(z_ref.at[zidx[b]], zbuf[b], semz[b])

    def wait_z(b):
        pltpu.make_async_copy(z_ref.at[zidx[b]], zbuf[b], semz[b]).wait()

    def compute(wt_row_fn, b):
        zb = zbuf[b]
        ab = aggbuf[b]

        def x_body(j, _):
            wrow = wt_row_fn(j)
            wsp = [wrow[t] for t in range(T)]
            sv = wsp[0]
            for t in range(1, T):
                sv = sv + wsp[t]
            inv = 1.0 / lax.broadcast(sv, (LANES,))
            row = j * T
            for d in range(nd):
                sl = pl.ds(d * LANES, LANES)
                acc = zb[row, sl] * wsp[0]
                for t in range(1, T):
                    acc = acc + zb[row + t, sl] * wsp[t]
                ab[j, sl] = acc * inv
            return 0

        lax.fori_loop(0, C, x_body, 0, unroll=False)

    na = pw // C

    def a_build_fire_z(c, b):
        r0 = c * C
        for j in range(C):
            zidx[b][pl.ds(j * T, T)] = nbid[r0 + j, :]
        fire_z(b)

    def a_fire_self(c, b):
        return pltpu.async_copy(h_ref.at[sid.at[pl.ds(c * C, C)]], xself[b],
                                semnb[b])

    def a_wait_self(b):
        pltpu.make_async_copy(h_ref.at[sid.at[pl.ds(0, C)]], xself[b],
                              semnb[b]).wait()

    a_build_fire_z(0, 0)
    a_fire_self(0, 0)

    def a_half(m, b):
        c = 2 * m + b
        other = 1 - b

        @pl.when(c + 1 < na)
        def _():
            a_build_fire_z(c + 1, other)
            a_fire_self(c + 1, other)

        wait_z(b)
        a_wait_self(b)
        compute(lambda j, _r0=c * C: swt[_r0 + j, :], b)
        o1 = pltpu.async_copy(xself[b], selfh_s.at[pl.ds(base + c * C, C)],
                              semo[b])
        o2 = pltpu.async_copy(aggbuf[b], agg_s.at[pl.ds(base + c * C, C)],
                              semo[b])
        o1.wait()
        o2.wait()
        return 0

    def a_iter(m, _):
        a_half(m, 0)
        a_half(m, 1)
        return 0

    lax.fori_loop(0, na // 2, a_iter, 0, unroll=False)

    nbch = (pw * T) // C

    def b_fire_nb(c, b):
        idxrow = nbid.at[c // 2, pl.ds((c % 2) * C, C)]
        pltpu.async_copy(nbn_ref.at[idxrow], xnb[b], semnb[b])
        pltpu.async_copy(nbw_ref.at[idxrow], xwt[b], semnb[b])
        pltpu.async_copy(h_ref.at[idxrow], xself[b], semnb[b])

    def b_wait_nb(b):
        idxrow = nbid.at[0, pl.ds(0, C)]
        pltpu.make_async_copy(nbn_ref.at[idxrow], xnb[b], semnb[b]).wait()
        pltpu.make_async_copy(nbw_ref.at[idxrow], xwt[b], semnb[b]).wait()
        pltpu.make_async_copy(h_ref.at[idxrow], xself[b], semnb[b]).wait()

    def b_build_fire_z(b):
        for j in range(C):
            zidx[b][pl.ds(j * T, T)] = xnb[b][j, :]
        fire_z(b)

    b_fire_nb(0, 0)
    b_fire_nb(1, 1)
    b_wait_nb(0)
    b_build_fire_z(0)

    def b_half(m, b):
        c = 2 * m + b
        other = 1 - b

        wait_z(b)

        @pl.when(c + 1 < nbch)
        def _():
            b_wait_nb(other)
            b_build_fire_z(other)

        compute(lambda j: xwt[b][j, :], b)
        sidx[0, :] = lax.iota(jnp.int32, LANES) * nb + (b * C * nb + base + c // 2)
        idx8 = sidx.at[0, pl.ds(0, C)]
        o1 = pltpu.async_copy(xself[b], selfh_n.at[idx8], semo[b])
        o2 = pltpu.async_copy(aggbuf[b], agg_n.at[idx8], semo[b])
        o1.wait()
        o2.wait()

        @pl.when(c + 2 < nbch)
        def _():
            b_fire_nb(c + 2, b)

        return 0

    def b_iter(m, _):
        b_half(m, 0)
        b_half(m, 1)
        return 0

    lax.fori_loop(0, nbch // 2, b_iter, 0, unroll=False)


def _sc_gather_agg(nodeset, nb_nodes_all, nb_weights_all, initial_h, z):
    nb = nodeset.shape[0]
    pw = nb // NW
    d_in = initial_h.shape[1]
    dh = z.shape[1]
    fn = pl.kernel(
        _sc_body,
        out_type=[
            jax.ShapeDtypeStruct((nb, d_in), jnp.float32),
            jax.ShapeDtypeStruct((nb, dh), jnp.float32),
            jax.ShapeDtypeStruct((nb * T, d_in), jnp.float32),
            jax.ShapeDtypeStruct((nb * T, dh), jnp.float32),
            jax.ShapeDtypeStruct((nb, T), jnp.float32),
        ],
        mesh=plsc.VectorSubcoreMesh(core_axis_name="c", subcore_axis_name="s"),
        compiler_params=pltpu.CompilerParams(use_tc_tiling_on_sc=False),
        scratch_types=[
            pltpu.VMEM((pw,), jnp.int32),
            pltpu.VMEM((pw, T), jnp.int32),
            pltpu.VMEM((pw, T), jnp.float32),
            pltpu.VMEM((C, T), jnp.int32),
            pltpu.VMEM((C, T), jnp.int32),
            pltpu.VMEM((C, T), jnp.float32),
            pltpu.VMEM((C, T), jnp.float32),
            pltpu.VMEM((C, d_in), jnp.float32),
            pltpu.VMEM((C, d_in), jnp.float32),
            pltpu.VMEM((C * T,), jnp.int32),
            pltpu.VMEM((C * T,), jnp.int32),
            pltpu.VMEM((C * T, dh), jnp.float32),
            pltpu.VMEM((C * T, dh), jnp.float32),
            pltpu.VMEM((C, dh), jnp.float32),
            pltpu.VMEM((C, dh), jnp.float32),
            pltpu.VMEM((1, LANES), jnp.int32),
            pltpu.SemaphoreType.DMA,
            pltpu.SemaphoreType.DMA,
            pltpu.SemaphoreType.DMA,
            pltpu.SemaphoreType.DMA,
            pltpu.SemaphoreType.DMA,
            pltpu.SemaphoreType.DMA,
        ],
    )
    return fn(nodeset, nb_nodes_all, nb_weights_all, initial_h, z)


def _mid_body(s_ref, a_ref, wa_ref, wb_ref, b_ref, o_ref):
    h = jnp.dot(s_ref[...], wa_ref[...], preferred_element_type=jnp.float32)
    h = h + jnp.dot(a_ref[...], wb_ref[...], preferred_element_type=jnp.float32)
    h = jnp.maximum(h + b_ref[...], 0.0)
    o_ref[...] = h / jnp.sqrt(jnp.sum(h * h, axis=1, keepdims=True))


def _mid_layer(selfh, agg, w0at, w0bt, w0b):
    n, d_in = selfh.shape
    dh = agg.shape[1]
    d_out = w0at.shape[1]
    blk = 512
    return pl.pallas_call(
        _mid_body,
        grid=(n // blk,),
        in_specs=[
            pl.BlockSpec((blk, d_in), lambda i: (i, 0)),
            pl.BlockSpec((blk, dh), lambda i: (i, 0)),
            pl.BlockSpec((d_in, d_out), lambda i: (0, 0)),
            pl.BlockSpec((dh, d_out), lambda i: (0, 0)),
            pl.BlockSpec((1, d_out), lambda i: (0, 0)),
        ],
        out_specs=pl.BlockSpec((blk, d_out), lambda i: (i, 0)),
        out_shape=jax.ShapeDtypeStruct((n, d_out), jnp.float32),
    )(selfh, agg, w0at, w0bt, w0b)


def _mid_body3(s_ref, a_ref, wa_ref, wb_ref, b_ref, o_ref):
    h = jnp.dot(s_ref[...], wa_ref[...], preferred_element_type=jnp.float32)
    h = h + jnp.dot(a_ref[...], wb_ref[...], preferred_element_type=jnp.float32)
    h = jnp.maximum(h + b_ref[...], 0.0)
    o_ref[0] = h / jnp.sqrt(jnp.sum(h * h, axis=1, keepdims=True))


def _mid_layer_nb(selfh, agg, w0at, w0bt, w0b, nb):
    n, d_in = selfh.shape
    dh = agg.shape[1]
    d_out = w0at.shape[1]
    blk = 512
    nblk = nb // blk
    return pl.pallas_call(
        _mid_body3,
        grid=(T, nblk),
        in_specs=[
            pl.BlockSpec((blk, d_in), lambda t, i: (t * nblk + i, 0)),
            pl.BlockSpec((blk, dh), lambda t, i: (t * nblk + i, 0)),
            pl.BlockSpec((d_in, d_out), lambda t, i: (0, 0)),
            pl.BlockSpec((dh, d_out), lambda t, i: (0, 0)),
            pl.BlockSpec((1, d_out), lambda t, i: (0, 0)),
        ],
        out_specs=pl.BlockSpec((1, blk, d_out), lambda t, i: (t, i, 0)),
        out_shape=jax.ShapeDtypeStruct((T, nb, d_out), jnp.float32),
    )(selfh, agg, w0at, w0bt, w0b)


def _fin_body(hs_ref, hn_ref, w_ref, q1w_ref, q1b_ref, w1a_ref, w1b_ref,
              w1bias_ref, g1w_ref, g1b_ref, g2w_ref, o_ref):
    wts = w_ref[...]
    blk = hs_ref.shape[0]
    dh = q1w_ref.shape[1]
    acc = jnp.zeros((blk, dh), jnp.float32)
    for t in range(T):
        nh = jnp.dot(hn_ref[t], q1w_ref[...], preferred_element_type=jnp.float32)
        nh = jnp.maximum(nh + q1b_ref[...], 0.0)
        acc = acc + nh * wts[:, t:t + 1]
    agg = acc / jnp.sum(wts, axis=1, keepdims=True)
    h = jnp.dot(hs_ref[...], w1a_ref[...], preferred_element_type=jnp.float32)
    h = h + jnp.dot(agg, w1b_ref[...], preferred_element_type=jnp.float32)
    h = jnp.maximum(h + w1bias_ref[...], 0.0)
    h = h / jnp.sqrt(jnp.sum(h * h, axis=1, keepdims=True))
    g = jnp.maximum(jnp.dot(h, g1w_ref[...], preferred_element_type=jnp.float32)
                    + g1b_ref[...], 0.0)
    o_ref[...] = jnp.dot(g, g2w_ref[...], preferred_element_type=jnp.float32)


def _final_layer(h1self, h1nb, nbw, q1wt, q1b, w1at, w1bt, w1b, g1wt, g1b, g2wt):
    nb, d = h1self.shape
    dh = q1wt.shape[1]
    d_out = g2wt.shape[1]
    blk = 512
    return pl.pallas_call(
        _fin_body,
        grid=(nb // blk,),
        in_specs=[
            pl.BlockSpec((blk, d), lambda i: (i, 0)),
            pl.BlockSpec((T, blk, d), lambda i: (0, i, 0)),
            pl.BlockSpec((blk, T), lambda i: (i, 0)),
            pl.BlockSpec((d, dh), lambda i: (0, 0)),
            pl.BlockSpec((1, dh), lambda i: (0, 0)),
            pl.BlockSpec((d, d), lambda i: (0, 0)),
            pl.BlockSpec((dh, d), lambda i: (0, 0)),
            pl.BlockSpec((1, d), lambda i: (0, 0)),
            pl.BlockSpec((d, d), lambda i: (0, 0)),
            pl.BlockSpec((1, d), lambda i: (0, 0)),
            pl.BlockSpec((d, d_out), lambda i: (0, 0)),
        ],
        out_specs=pl.BlockSpec((blk, d_out), lambda i: (i, 0)),
        out_shape=jax.ShapeDtypeStruct((nb, d_out), jnp.float32),
    )(h1self, h1nb, nbw, q1wt, q1b, w1at, w1bt, w1b, g1wt, g1b, g2wt)


def kernel(initial_h, nodeset, nb_weights_all, nb_nodes_all,
           Q0_w, Q0_b, W0_w, W0_b, Q1_w, Q1_b, W1_w, W1_b, G1_w, G1_b, G2_w):
    nb = nodeset.shape[0]
    d_in = initial_h.shape[1]

    nodeset = nodeset.astype(jnp.int32)
    nb_nodes = nb_nodes_all[:, :T].astype(jnp.int32)
    nb_weights = nb_weights_all[:, :T]

    z = _z_table(initial_h, Q0_w.T, Q0_b.reshape(1, -1))

    selfh_s, agg_s, selfh_n, agg_n, nbw = _sc_gather_agg(
        nodeset, nb_nodes, nb_weights, initial_h, z)

    w0at = W0_w[:, :d_in].T
    w0bt = W0_w[:, d_in:].T
    w0b = W0_b.reshape(1, -1)

    h1self = _mid_layer(selfh_s, agg_s, w0at, w0bt, w0b)
    h1nb = _mid_layer_nb(selfh_n, agg_n, w0at, w0bt, w0b, nb)

    d1 = h1self.shape[1]
    return _final_layer(h1self, h1nb, nbw,
                        Q1_w.T, Q1_b.reshape(1, -1),
                        W1_w[:, :d1].T, W1_w[:, d1:].T, W1_b.reshape(1, -1),
                        G1_w.T, G1_b.reshape(1, -1), G2_w.T)


# speedup vs baseline: 1.3847x; 1.3847x over previous
import functools

import jax
import jax.numpy as jnp
from jax import lax
from jax.experimental import pallas as pl
from jax.experimental.pallas import tpu as pltpu
from jax.experimental.pallas import tpu_sc as plsc

T = 16
NW = 32
LANES = 16
C = 8


def _ztab_body(x_ref, w_ref, b_ref, o_ref):
    acc = jnp.dot(x_ref[...], w_ref[...], preferred_element_type=jnp.float32)
    o_ref[...] = jnp.maximum(acc + b_ref[...], 0.0)


def _z_table(h, q0wt, q0b):
    n, d = h.shape
    dh = q0wt.shape[1]
    blk = 1000
    return pl.pallas_call(
        _ztab_body,
        grid=(n // blk,),
        in_specs=[
            pl.BlockSpec((blk, d), lambda i: (i, 0)),
            pl.BlockSpec((d, dh), lambda i: (0, 0)),
            pl.BlockSpec((1, dh), lambda i: (0, 0)),
        ],
        out_specs=pl.BlockSpec((blk, dh), lambda i: (i, 0)),
        out_shape=jax.ShapeDtypeStruct((n, dh), jnp.float32),
    )(h, q0wt, q0b)


def _sc1_body(nbn_ref, nbw_ref, z_ref, agg_out,
              nbid0, nbid1, wt0, wt1, zidx0, zidx1, zbuf0, zbuf1,
              aggbuf0, aggbuf1,
              semnb0, semnb1, semz0, semz1, semo0, semo1):
    nrows = nbn_ref.shape[0]
    nch = nrows // C
    per_w = (nch + NW - 1) // NW
    dh = z_ref.shape[1]
    nd = dh // LANES
    wid = lax.axis_index("s") * 2 + lax.axis_index("c")
    base_ch = wid * per_w

    nbid = (nbid0, nbid1)
    wt = (wt0, wt1)
    zidx = (zidx0, zidx1)
    zbuf = (zbuf0, zbuf1)
    aggbuf = (aggbuf0, aggbuf1)
    semnb = (semnb0, semnb1)
    semz = (semz0, semz1)
    semo = (semo0, semo1)

    def c_eff(c):
        return jnp.minimum(base_ch + c, nch - 1)

    def fire_nb(c, b):
        r0 = c_eff(c) * C
        pltpu.async_copy(nbn_ref.at[pl.ds(r0, C)], nbid[b], semnb[b])
        pltpu.async_copy(nbw_ref.at[pl.ds(r0, C)], wt[b], semnb[b])

    def wait_nb(b):
        pltpu.make_async_copy(nbn_ref.at[pl.ds(0, C)], nbid[b], semnb[b]).wait()
        pltpu.make_async_copy(nbw_ref.at[pl.ds(0, C)], wt[b], semnb[b]).wait()

    def build_fire_z(b):
        for j in range(C):
            zidx[b][pl.ds(j * T, T)] = nbid[b][j, :]
        pltpu.async_copy(z_ref.at[zidx[b]], zbuf[b], semz[b])

    def wait_z(b):
        pltpu.make_async_copy(z_ref.at[zidx[b]], zbuf[b], semz[b]).wait()

    def compute(b):
        zb = zbuf[b]
        wb = wt[b]
        ab = aggbuf[b]

        def x_body(j, _):
            wrow = wb[j, :]
            wsp = [wrow[t] for t in range(T)]
            sv = wsp[0]
            for t in range(1, T):
                sv = sv + wsp[t]
            inv = 1.0 / lax.broadcast(sv, (LANES,))
            row = j * T
            for d in range(nd):
                sl = pl.ds(d * LANES, LANES)
                acc = zb[row, sl] * wsp[0]
                for t in range(1, T):
                    acc = acc + zb[row + t, sl] * wsp[t]
                ab[j, sl] = acc * inv
            return 0

        lax.fori_loop(0, C, x_body, 0, unroll=False)

    fire_nb(0, 0)
    fire_nb(1, 1)
    wait_nb(0)
    build_fire_z(0)

    def half(m, b):
        c = 2 * m + b
        other = 1 - b

        wait_z(b)

        @pl.when(c + 1 < per_w)
        def _():
            wait_nb(other)
            build_fire_z(other)

        compute(b)
        r0 = c_eff(c) * C
        pltpu.async_copy(aggbuf[b], agg_out.at[pl.ds(r0, C)], semo[b]).wait()

        @pl.when(c + 2 < per_w)
        def _():
            fire_nb(c + 2, b)

        return 0

    def it(m, _):
        half(m, 0)
        half(m, 1)
        return 0

    lax.fori_loop(0, per_w // 2, it, 0, unroll=False)


def _sc_dense_agg(nbn, nbw, z):
    n = nbn.shape[0]
    dh = z.shape[1]
    fn = pl.kernel(
        _sc1_body,
        out_type=[jax.ShapeDtypeStruct((n, dh), jnp.float32)],
        mesh=plsc.VectorSubcoreMesh(core_axis_name="c", subcore_axis_name="s"),
        compiler_params=pltpu.CompilerParams(use_tc_tiling_on_sc=False),
        scratch_types=[
            pltpu.VMEM((C, T), jnp.int32),
            pltpu.VMEM((C, T), jnp.int32),
            pltpu.VMEM((C, T), jnp.float32),
            pltpu.VMEM((C, T), jnp.float32),
            pltpu.VMEM((C * T,), jnp.int32),
            pltpu.VMEM((C * T,), jnp.int32),
            pltpu.VMEM((C * T, dh), jnp.float32),
            pltpu.VMEM((C * T, dh), jnp.float32),
            pltpu.VMEM((C, dh), jnp.float32),
            pltpu.VMEM((C, dh), jnp.float32),
            pltpu.SemaphoreType.DMA,
            pltpu.SemaphoreType.DMA,
            pltpu.SemaphoreType.DMA,
            pltpu.SemaphoreType.DMA,
            pltpu.SemaphoreType.DMA,
            pltpu.SemaphoreType.DMA,
        ],
    )
    return fn(nbn, nbw, z)[0]


def _sc2_body(ns_ref, nbn_ref, nbw_ref, h1_ref,
              h1self_out, h1nb_out, nbw_out,
              sid, nbid, swt, selfbuf, nbflat0, nbflat1, nbbuf0, nbbuf1, sidx,
              semg, semz0, semz1, semo0, semo1):
    nb = ns_ref.shape[0]
    pw = nb // NW
    wid = lax.axis_index("s") * 2 + lax.axis_index("c")
    base = wid * pw

    nbflat = (nbflat0, nbflat1)
    nbbuf = (nbbuf0, nbbuf1)
    semz = (semz0, semz1)
    semo = (semo0, semo1)

    pltpu.sync_copy(ns_ref.at[pl.ds(base, pw)], sid)
    pltpu.async_copy(nbn_ref.at[sid], nbid, semg).wait()
    pltpu.async_copy(nbw_ref.at[sid], swt, semg).wait()
    pltpu.sync_copy(swt, nbw_out.at[pl.ds(base, pw)])

    pltpu.async_copy(h1_ref.at[sid], selfbuf, semg).wait()
    pltpu.async_copy(selfbuf, h1self_out.at[pl.ds(base, pw)], semg).wait()

    nch = pw // C

    def build_fire(c, b):
        r0 = c * C
        for j in range(C):
            nbflat[b][pl.ds(j * T, T)] = nbid[r0 + j, :]
        pltpu.async_copy(h1_ref.at[nbflat[b]], nbbuf[b], semz[b])

    def wait_g(b):
        pltpu.make_async_copy(h1_ref.at[nbflat[b]], nbbuf[b], semz[b]).wait()

    build_fire(0, 0)
    build_fire(1, 1)

    def half(m, b):
        c = 2 * m + b

        wait_g(b)
        for j in range(C):
            sidx[pl.ds(j * T, T)] = (lax.iota(jnp.int32, T) * nb
                                     + (base + c * C + j))
        pltpu.async_copy(nbbuf[b], h1nb_out.at[sidx], semo[b]).wait()

        @pl.when(c + 2 < nch)
        def _():
            build_fire(c + 2, b)

        return 0

    def it(m, _):
        half(m, 0)
        half(m, 1)
        return 0

    lax.fori_loop(0, nch // 2, it, 0, unroll=False)


def _sc_h1_gather(nodeset, nbn, nbw, h1):
    nb = nodeset.shape[0]
    pw = nb // NW
    d1 = h1.shape[1]
    fn = pl.kernel(
        _sc2_body,
        out_type=[
            jax.ShapeDtypeStruct((nb, d1), jnp.float32),
            jax.ShapeDtypeStruct((nb * T, d1), jnp.float32),
            jax.ShapeDtypeStruct((nb, T), jnp.float32),
        ],
        mesh=plsc.VectorSubcoreMesh(core_axis_name="c", subcore_axis_name="s"),
        compiler_params=pltpu.CompilerParams(use_tc_tiling_on_sc=False),
        scratch_types=[
            pltpu.VMEM((pw,), jnp.int32),
            pltpu.VMEM((pw, T), jnp.int32),
            pltpu.VMEM((pw, T), jnp.float32),
            pltpu.VMEM((pw, d1), jnp.float32),
            pltpu.VMEM((C * T,), jnp.int32),
            pltpu.VMEM((C * T,), jnp.int32),
            pltpu.VMEM((C * T, d1), jnp.float32),
            pltpu.VMEM((C * T, d1), jnp.float32),
            pltpu.VMEM((C * T,), jnp.int32),
            pltpu.SemaphoreType.DMA,
            pltpu.SemaphoreType.DMA,
            pltpu.SemaphoreType.DMA,
            pltpu.SemaphoreType.DMA,
            pltpu.SemaphoreType.DMA,
        ],
    )
    return fn(nodeset, nbn, nbw, h1)


def _mid_body(s_ref, a_ref, wa_ref, wb_ref, b_ref, o_ref):
    h = jnp.dot(s_ref[...], wa_ref[...], preferred_element_type=jnp.float32)
    h = h + jnp.dot(a_ref[...], wb_ref[...], preferred_element_type=jnp.float32)
    h = jnp.maximum(h + b_ref[...], 0.0)
    o_ref[...] = h / jnp.sqrt(jnp.sum(h * h, axis=1, keepdims=True))


def _mid_layer(selfh, agg, w0at, w0bt, w0b):
    n, d_in = selfh.shape
    dh = agg.shape[1]
    d_out = w0at.shape[1]
    blk = 2000
    return pl.pallas_call(
        _mid_body,
        grid=(n // blk,),
        in_specs=[
            pl.BlockSpec((blk, d_in), lambda i: (i, 0)),
            pl.BlockSpec((blk, dh), lambda i: (i, 0)),
            pl.BlockSpec((d_in, d_out), lambda i: (0, 0)),
            pl.BlockSpec((dh, d_out), lambda i: (0, 0)),
            pl.BlockSpec((1, d_out), lambda i: (0, 0)),
        ],
        out_specs=pl.BlockSpec((blk, d_out), lambda i: (i, 0)),
        out_shape=jax.ShapeDtypeStruct((n, d_out), jnp.float32),
    )(selfh, agg, w0at, w0bt, w0b)


def _fin_body(hs_ref, hn_ref, w_ref, q1w_ref, q1b_ref, w1a_ref, w1b_ref,
              w1bias_ref, g1w_ref, g1b_ref, g2w_ref, o_ref):
    wts = w_ref[...]
    blk = hs_ref.shape[0]
    dh = q1w_ref.shape[1]
    acc = jnp.zeros((blk, dh), jnp.float32)
    for t in range(T):
        nh = jnp.dot(hn_ref[t], q1w_ref[...], preferred_element_type=jnp.float32)
        nh = jnp.maximum(nh + q1b_ref[...], 0.0)
        acc = acc + nh * wts[:, t:t + 1]
    agg = acc / jnp.sum(wts, axis=1, keepdims=True)
    h = jnp.dot(hs_ref[...], w1a_ref[...], preferred_element_type=jnp.float32)
    h = h + jnp.dot(agg, w1b_ref[...], preferred_element_type=jnp.float32)
    h = jnp.maximum(h + w1bias_ref[...], 0.0)
    h = h / jnp.sqrt(jnp.sum(h * h, axis=1, keepdims=True))
    g = jnp.maximum(jnp.dot(h, g1w_ref[...], preferred_element_type=jnp.float32)
                    + g1b_ref[...], 0.0)
    o_ref[...] = jnp.dot(g, g2w_ref[...], preferred_element_type=jnp.float32)


def _final_layer(h1self, h1nb, nbw, q1wt, q1b, w1at, w1bt, w1b, g1wt, g1b, g2wt):
    nb, d = h1self.shape
    dh = q1wt.shape[1]
    d_out = g2wt.shape[1]
    blk = 512
    return pl.pallas_call(
        _fin_body,
        grid=(nb // blk,),
        in_specs=[
            pl.BlockSpec((blk, d), lambda i: (i, 0)),
            pl.BlockSpec((T, blk, d), lambda i: (0, i, 0)),
            pl.BlockSpec((blk, T), lambda i: (i, 0)),
            pl.BlockSpec((d, dh), lambda i: (0, 0)),
            pl.BlockSpec((1, dh), lambda i: (0, 0)),
            pl.BlockSpec((d, d), lambda i: (0, 0)),
            pl.BlockSpec((dh, d), lambda i: (0, 0)),
            pl.BlockSpec((1, d), lambda i: (0, 0)),
            pl.BlockSpec((d, d), lambda i: (0, 0)),
            pl.BlockSpec((1, d), lambda i: (0, 0)),
            pl.BlockSpec((d, d_out), lambda i: (0, 0)),
        ],
        out_specs=pl.BlockSpec((blk, d_out), lambda i: (i, 0)),
        out_shape=jax.ShapeDtypeStruct((nb, d_out), jnp.float32),
    )(h1self, h1nb, nbw, q1wt, q1b, w1at, w1bt, w1b, g1wt, g1b, g2wt)


def kernel(initial_h, nodeset, nb_weights_all, nb_nodes_all,
           Q0_w, Q0_b, W0_w, W0_b, Q1_w, Q1_b, W1_w, W1_b, G1_w, G1_b, G2_w):
    nb = nodeset.shape[0]
    d_in = initial_h.shape[1]

    nodeset = nodeset.astype(jnp.int32)
    nbn = nb_nodes_all[:, :T].astype(jnp.int32)
    nbw = nb_weights_all[:, :T]

    z = _z_table(initial_h, Q0_w.T, Q0_b.reshape(1, -1))

    agg = _sc_dense_agg(nbn, nbw, z)

    w0at = W0_w[:, :d_in].T
    w0bt = W0_w[:, d_in:].T
    w0b = W0_b.reshape(1, -1)

    h1 = _mid_layer(initial_h, agg, w0at, w0bt, w0b)

    h1self, h1nb_flat, nbw_ns = _sc_h1_gather(nodeset, nbn, nbw, h1)
    d1 = h1.shape[1]
    h1nb = h1nb_flat.reshape(T, nb, d1)

    return _final_layer(h1self, h1nb, nbw_ns,
                        Q1_w.T, Q1_b.reshape(1, -1),
                        W1_w[:, :d1].T, W1_w[:, d1:].T, W1_b.reshape(1, -1),
                        G1_w.T, G1_b.reshape(1, -1), G2_w.T)
